# Initial kernel scaffold; baseline (speedup 1.0000x reference)
import jax
import jax.numpy as jnp
from jax.experimental import pallas as pl


def kernel(points, point_features, W1, b1, g1, be1, W2, b2, g2, be2, W3, b3):
    raise NotImplementedError("write your pallas kernel here")


# speedup vs baseline: 12.1076x; 12.1076x over previous
import functools

import jax
import jax.numpy as jnp
from jax import lax
from jax.experimental import pallas as pl
from jax.experimental.pallas import tpu as pltpu
from jax.experimental.pallas import tpu_sc as plsc

B = 8
N = 2048
M = N // 4
G = 32
C = 64
CO = 128
R2 = 0.2 * 0.2
S = B * M * G
_HI = lax.Precision.HIGHEST


def _fps_body(x_ref, y_ref, z_ref, cx_ref, cy_ref, cz_ref):
    x = x_ref[...]
    y = y_ref[...]
    z = z_ref[...]
    iota = lax.broadcasted_iota(jnp.int32, (B, N), 1)

    miota = lax.broadcasted_iota(jnp.int32, (B, M), 1)

    px = x[:, 0:1]
    py = y[:, 0:1]
    pz = z[:, 0:1]
    cxs = jnp.broadcast_to(px, (B, M))
    cys = jnp.broadcast_to(py, (B, M))
    czs = jnp.broadcast_to(pz, (B, M))

    def body(i, carry):
        dists, px, py, pz, cxs, cys, czs = carry
        dx = x - px
        dy = y - py
        dz = z - pz
        d = (dx * dx + dy * dy) + dz * dz
        dists = jnp.minimum(dists, d)
        m = jnp.max(dists, axis=1, keepdims=True)
        nidx = jnp.min(jnp.where(dists == m, iota, N), axis=1, keepdims=True)
        sel = iota == nidx
        npx = jnp.sum(jnp.where(sel, x, 0.0), axis=1, keepdims=True)
        npy = jnp.sum(jnp.where(sel, y, 0.0), axis=1, keepdims=True)
        npz = jnp.sum(jnp.where(sel, z, 0.0), axis=1, keepdims=True)
        put = miota == i
        cxs = jnp.where(put, npx, cxs)
        cys = jnp.where(put, npy, cys)
        czs = jnp.where(put, npz, czs)
        return (dists, npx, npy, npz, cxs, cys, czs)

    out = lax.fori_loop(1, M, body,
                        (jnp.full((B, N), 1e10, jnp.float32), px, py, pz,
                         cxs, cys, czs))
    cx_ref[...] = out[4]
    cy_ref[...] = out[5]
    cz_ref[...] = out[6]


def _fps(x, y, z):
    out = jax.ShapeDtypeStruct((B, M), jnp.float32)
    return pl.pallas_call(
        _fps_body,
        out_shape=(out, out, out),
    )(x, y, z)


_RT = 64
_NT = M // _RT


def _ball_body(x_ref, y_ref, z_ref, ccx_ref, ccy_ref, ccz_ref, idx_ref):
    b = pl.program_id(0)
    x = x_ref[0]
    y = y_ref[0]
    z = z_ref[0]
    cxc = ccx_ref[0]
    cyc = ccy_ref[0]
    czc = ccz_ref[0]

    dx = cxc - x
    dy = cyc - y
    dz = czc - z
    d2 = (dx * dx + dy * dy) + dz * dz
    rank = jnp.where(d2 < R2, 1.0, 0.0)
    s = 1
    while s < N:
        shifted = jnp.concatenate(
            [jnp.zeros((_RT, s), jnp.float32), rank[:, :N - s]], axis=1)
        rank = rank + shifted
        s *= 2
    count = rank[:, N - 1:N]

    cols = []
    for g in range(G):
        cmp = jnp.where(rank <= float(g), 1.0, 0.0)
        cols.append(jnp.sum(cmp, axis=1, keepdims=True))
    pos = jnp.concatenate(cols, axis=1)
    gio = lax.broadcasted_iota(jnp.int32, (_RT, G), 1)
    pos = jnp.where(gio < count.astype(jnp.int32), pos, pos[:, 0:1])
    idx_ref[...] = (pos.astype(jnp.int32) + b * N)[None]


def _ball(x, y, z, ccx, ccy, ccz):
    return pl.pallas_call(
        _ball_body,
        grid=(B, _NT),
        in_specs=[
            pl.BlockSpec((1, 1, N), lambda b, t: (b, 0, 0)),
            pl.BlockSpec((1, 1, N), lambda b, t: (b, 0, 0)),
            pl.BlockSpec((1, 1, N), lambda b, t: (b, 0, 0)),
            pl.BlockSpec((1, _RT, 1), lambda b, t: (b, t, 0)),
            pl.BlockSpec((1, _RT, 1), lambda b, t: (b, t, 0)),
            pl.BlockSpec((1, _RT, 1), lambda b, t: (b, t, 0)),
        ],
        out_specs=pl.BlockSpec((1, _RT, G), lambda b, t: (b, t, 0)),
        out_shape=jax.ShapeDtypeStruct((B, M, G), jnp.int32),
    )(x, y, z, ccx, ccy, ccz)


def _tab_body(xcat_ref, w1t_ref, b1_ref, ccx_ref, ccy_ref, ccz_ref,
              a1_ref, q1_ref):
    w1t = w1t_ref[...]
    a1 = jnp.dot(xcat_ref[0], w1t, preferred_element_type=jnp.float32,
                 precision=_HI) + b1_ref[...]
    a1_ref[0] = a1
    cxc = ccx_ref[0]
    cyc = ccy_ref[0]
    czc = ccz_ref[0]
    q1_ref[0] = cxc * w1t[0:1, :] + cyc * w1t[1:2, :] + czc * w1t[2:3, :]


def _tab(xcat, w1t, b1, ccx, ccy, ccz):
    return pl.pallas_call(
        _tab_body,
        grid=(B,),
        in_specs=[
            pl.BlockSpec((1, N, C + 3), lambda b: (b, 0, 0)),
            pl.BlockSpec((C + 3, C), lambda b: (0, 0)),
            pl.BlockSpec((1, C), lambda b: (0, 0)),
            pl.BlockSpec((1, M, 1), lambda b: (b, 0, 0)),
            pl.BlockSpec((1, M, 1), lambda b: (b, 0, 0)),
            pl.BlockSpec((1, M, 1), lambda b: (b, 0, 0)),
        ],
        out_specs=[
            pl.BlockSpec((1, N, C), lambda b: (b, 0, 0)),
            pl.BlockSpec((1, M, C), lambda b: (b, 0, 0)),
        ],
        out_shape=[
            jax.ShapeDtypeStruct((B, N, C), jnp.float32),
            jax.ShapeDtypeStruct((B, M, C), jnp.float32),
        ],
    )(xcat, w1t, b1, ccx, ccy, ccz)


_NW = 32
_CH = 128
_NCHUNK = S // _NW // _CH


def _gather_sc(table, idx3):
    mesh = plsc.VectorSubcoreMesh(core_axis_name="c", subcore_axis_name="s")

    @functools.partial(
        pl.kernel,
        out_type=jax.ShapeDtypeStruct((S, C), jnp.float32),
        mesh=mesh,
        scratch_types=[
            pltpu.VMEM((_NCHUNK, _CH), jnp.int32),
            pltpu.VMEM((_CH, C), jnp.float32),
            pltpu.VMEM((_CH, C), jnp.float32),
            pltpu.SemaphoreType.DMA,
            pltpu.SemaphoreType.DMA,
        ],
        compiler_params=pltpu.CompilerParams(use_tc_tiling_on_sc=False),
    )
    def gather_k(table_hbm, idx_hbm, out_hbm, idx_v, buf0, buf1, sem0, sem1):
        wid = lax.axis_index("s") * 2 + lax.axis_index("c")
        base = wid * (_NCHUNK * _CH)
        pltpu.sync_copy(idx_hbm.at[wid], idx_v)
        cp0 = pltpu.async_copy(table_hbm.at[idx_v.at[0]], buf0, sem0)

        def body(c, _):
            even = lax.rem(c, 2) == 0

            @pl.when(jnp.logical_and(even, c + 1 < _NCHUNK))
            def _():
                pltpu.async_copy(table_hbm.at[idx_v.at[c + 1]], buf1, sem1)

            @pl.when(jnp.logical_and(jnp.logical_not(even), c + 1 < _NCHUNK))
            def _():
                pltpu.async_copy(table_hbm.at[idx_v.at[c + 1]], buf0, sem0)

            @pl.when(even)
            def _():
                pltpu.make_async_copy(table_hbm.at[idx_v.at[c]], buf0,
                                      sem0).wait()
                pltpu.sync_copy(buf0, out_hbm.at[pl.ds(base + c * _CH, _CH)])

            @pl.when(jnp.logical_not(even))
            def _():
                pltpu.make_async_copy(table_hbm.at[idx_v.at[c]], buf1,
                                      sem1).wait()
                pltpu.sync_copy(buf1, out_hbm.at[pl.ds(base + c * _CH, _CH)])

            return 0

        del cp0
        lax.fori_loop(0, _NCHUNK, body, 0)

    return gather_k(table, idx3)


_BLK = B * M * G // 64
_NB = S // _BLK
_GPB = _BLK // G


def _stats1_body(xg_ref, q1_ref, out_ref):
    i = pl.program_id(0)
    v = xg_ref[...].reshape(_GPB, G, C) - q1_ref[...][:, None, :]
    s = jnp.sum(v, axis=(0, 1))
    sq = jnp.sum(v * v, axis=(0, 1))
    upd = jnp.concatenate(
        [s[None], sq[None], jnp.zeros((6, C), jnp.float32)], axis=0)

    @pl.when(i == 0)
    def _():
        out_ref[...] = jnp.zeros((8, C), jnp.float32)

    out_ref[...] += upd


def _stats1(xg, q1f):
    return pl.pallas_call(
        _stats1_body,
        grid=(_NB,),
        in_specs=[
            pl.BlockSpec((_BLK, C), lambda i: (i, 0)),
            pl.BlockSpec((_GPB, C), lambda i: (i, 0)),
        ],
        out_specs=pl.BlockSpec((8, C), lambda i: (0, 0)),
        out_shape=jax.ShapeDtypeStruct((8, C), jnp.float32),
    )(xg, q1f)


def _pass2_body(xg_ref, q1_ref, a1_ref, c1_ref, w2t_ref, b2_ref,
                y2_ref, out_ref):
    i = pl.program_id(0)
    v = xg_ref[...].reshape(_GPB, G, C) - q1_ref[...][:, None, :]
    z1 = jnp.maximum(v.reshape(_BLK, C) * a1_ref[...] + c1_ref[...], 0.0)
    y2 = jnp.dot(z1, w2t_ref[...], preferred_element_type=jnp.float32,
                 precision=_HI) + b2_ref[...]
    y2_ref[...] = y2
    s = jnp.sum(y2, axis=0)
    sq = jnp.sum(y2 * y2, axis=0)
    upd = jnp.concatenate(
        [s[None], sq[None], jnp.zeros((6, C), jnp.float32)], axis=0)

    @pl.when(i == 0)
    def _():
        out_ref[...] = jnp.zeros((8, C), jnp.float32)

    out_ref[...] += upd


def _pass2(xg, q1f, a1c, c1c, w2t, b2):
    return pl.pallas_call(
        _pass2_body,
        grid=(_NB,),
        in_specs=[
            pl.BlockSpec((_BLK, C), lambda i: (i, 0)),
            pl.BlockSpec((_GPB, C), lambda i: (i, 0)),
            pl.BlockSpec((1, C), lambda i: (0, 0)),
            pl.BlockSpec((1, C), lambda i: (0, 0)),
            pl.BlockSpec((C, C), lambda i: (0, 0)),
            pl.BlockSpec((1, C), lambda i: (0, 0)),
        ],
        out_specs=[
            pl.BlockSpec((_BLK, C), lambda i: (i, 0)),
            pl.BlockSpec((8, C), lambda i: (0, 0)),
        ],
        out_shape=[
            jax.ShapeDtypeStruct((S, C), jnp.float32),
            jax.ShapeDtypeStruct((8, C), jnp.float32),
        ],
    )(xg, q1f, a1c, c1c, w2t, b2)


def _pass3_body(y2_ref, a2_ref, c2_ref, w3t_ref, b3_ref, out_ref):
    z2 = jnp.maximum(y2_ref[...] * a2_ref[...] + c2_ref[...], 0.0)
    y3 = jnp.dot(z2, w3t_ref[...], preferred_element_type=jnp.float32,
                 precision=_HI) + b3_ref[...]
    out_ref[...] = jnp.max(y3.reshape(_GPB, G, CO), axis=1)


def _pass3(y2, a2c, c2c, w3t, b3):
    return pl.pallas_call(
        _pass3_body,
        grid=(_NB,),
        in_specs=[
            pl.BlockSpec((_BLK, C), lambda i: (i, 0)),
            pl.BlockSpec((1, C), lambda i: (0, 0)),
            pl.BlockSpec((1, C), lambda i: (0, 0)),
            pl.BlockSpec((C, CO), lambda i: (0, 0)),
            pl.BlockSpec((1, CO), lambda i: (0, 0)),
        ],
        out_specs=pl.BlockSpec((_GPB, CO), lambda i: (i, 0)),
        out_shape=jax.ShapeDtypeStruct((B * M, CO), jnp.float32),
    )(y2, a2c, c2c, w3t, b3)


def kernel(points, point_features, W1, b1, g1, be1, W2, b2, g2, be2, W3, b3):
    x = points[:, 0, :]
    y = points[:, 1, :]
    z = points[:, 2, :]

    cx, cy, cz = _fps(x, y, z)

    pts_t = jnp.transpose(points, (0, 2, 1))
    feats_t = jnp.transpose(point_features, (0, 2, 1))
    xcat = jnp.concatenate([pts_t, feats_t], axis=-1)
    w1t = W1.T

    ccx, ccy, ccz = cx[..., None], cy[..., None], cz[..., None]
    idx = _ball(x[:, None, :], y[:, None, :], z[:, None, :], ccx, ccy, ccz)
    a1, q1 = _tab(xcat, w1t, b1[None, :], ccx, ccy, ccz)

    xg = _gather_sc(a1.reshape(B * N, C),
                    idx.reshape(_NW, _NCHUNK, _CH))

    q1f = q1.reshape(B * M, C)
    st1 = _stats1(xg, q1f)
    inv_s = 1.0 / S
    mu1 = st1[0] * inv_s
    var1 = st1[1] * inv_s - mu1 * mu1
    a1c = g1 * lax.rsqrt(var1 + 1e-5)
    c1c = be1 - mu1 * a1c

    y2, st2 = _pass2(xg, q1f, a1c[None], c1c[None], W2.T, b2[None])
    mu2 = st2[0] * inv_s
    var2 = st2[1] * inv_s - mu2 * mu2
    a2c = g2 * lax.rsqrt(var2 + 1e-5)
    c2c = be2 - mu2 * a2c

    feat_out = _pass3(y2, a2c[None], c2c[None], W3.T, b3[None])

    centroids = jnp.stack([cx, cy, cz], axis=-1).reshape(B, 3, M)
    centroid_features = jnp.transpose(feat_out.reshape(B, M, CO), (0, 2, 1))
    return (centroids, centroid_features)
